# baseline (speedup 1.0000x reference)
import functools

import numpy as np
import jax
import jax.numpy as jnp
from jax import lax
from jax.experimental import pallas as pl
from jax.experimental.pallas import tpu as pltpu
from jax.experimental.pallas import tpu_sc as plsc

_B = 64
_N = 196
_NR = 49
_LN = 512
_NRN = 76
_D = 768

_NC = 2
_NS = 16
_NW = _NC * _NS
_BPW = _B // _NW
_CH = _D // 16
_KI = 56
_KN = 80
_PAD_I = _KI - _NR
_PAD_N = _KN - _NRN


def _sinusoidal_pe(d_model, max_len):
    position = np.arange(max_len, dtype=np.float32)[:, None]
    div_term = np.exp(
        np.arange(0, d_model, 2, dtype=np.float32) * (-np.log(10000.0) / d_model))
    pe = np.zeros((max_len, d_model), dtype=np.float32)
    pe[:, 0::2] = np.sin(position * div_term)
    pe[:, 1::2] = np.cos(position * div_term)
    return pe


_PE_NLP = _sinusoidal_pe(_D, _LN)
_WIMG = (_B * np.concatenate([np.arange(_NR), np.arange(_KI - _NR)])
         ).astype(np.int32)
_WNLP = (_B * np.concatenate([np.arange(_NRN), np.arange(_KN - _NRN)])
         ).astype(np.int32)
_BPB = 8


def _rank_body(noise_ref, noiset_ref,
               rev_ref, shuf_ref, fimg_ref):
    g = pl.program_id(0)
    ii = lax.broadcasted_iota(jnp.int32, (_N, _N), 0)
    jj = lax.broadcasted_iota(jnp.int32, (_N, _N), 1)
    for i in range(_BPB):
        b = g * _BPB + i
        nrow = noise_ref[i]
        ncol = noiset_ref[i]
        m = ((nrow < ncol) | ((nrow == ncol) & (jj < ii))).astype(jnp.int32)
        rank_col = jnp.sum(m, axis=1, keepdims=True)
        rank_row = (_N - 1) - jnp.sum(m, axis=0, keepdims=True)
        rev_ref[i] = rank_row
        oneh = (rank_col == jj).astype(jnp.int32)
        shuf = jnp.sum(oneh * ii, axis=0, keepdims=True)
        shuf_ref[i] = shuf
        spad = jnp.concatenate([shuf[:, :_NR], shuf[:, :_PAD_I]], axis=1)
        fimg_ref[i] = spad * _B + b


_rank_call = pl.pallas_call(
    _rank_body,
    grid=(_B // _BPB,),
    in_specs=[
        pl.BlockSpec((_BPB, 1, _N), lambda b: (b, 0, 0)),
        pl.BlockSpec((_BPB, _N, 1), lambda b: (b, 0, 0)),
    ],
    out_specs=[
        pl.BlockSpec((_BPB, 1, _N), lambda b: (b, 0, 0)),
        pl.BlockSpec((_BPB, 1, _N), lambda b: (b, 0, 0)),
        pl.BlockSpec((_BPB, 1, _KI), lambda b: (b, 0, 0)),
    ],
    out_shape=[
        jax.ShapeDtypeStruct((_B, 1, _N), jnp.int32),
        jax.ShapeDtypeStruct((_B, 1, _N), jnp.int32),
        jax.ShapeDtypeStruct((_B, 1, _KI), jnp.int32),
    ],
)


def _chunk_offsets(n):
    offs = list(range(0, n - 15, 16))
    if offs[-1] + 16 < n:
        offs.append(n - 16)
    return offs


def _add_rows(k, val, pos):
    @pl.loop(0, k)
    def _addrow(r):
        for c in range(_CH):
            sl = pl.ds(c * 16, 16)
            plsc.addupdate(val.at[r, sl], pos[r, sl])


def _img_body(img_flat, pos_img, rimg, wimg,
              oimg, ip, ipr, iw, ib, sem1, sem2):
    wid = lax.axis_index("s") * _NC + lax.axis_index("c")
    pltpu.sync_copy(wimg, ib)

    for bb in range(_BPW):
        b = wid * _BPW + bb
        pltpu.sync_copy(rimg.at[b].at[0], ip)
        for o in _chunk_offsets(_KI):
            sl = pl.ds(o, 16)
            iw[sl] = ib[sl] + b
            ipr[sl] = lax.shift_right_logical(ip[sl], 6)

        def phase(val, pos):
            c1 = pltpu.async_copy(img_flat.at[ip], val, sem1)
            c2 = pltpu.async_copy(pos_img.at[ipr], pos, sem2)
            c1.wait()
            c2.wait()
            _add_rows(_KI, val, pos)
            pltpu.async_copy(val, oimg.at[iw], sem1).wait()

        pl.run_scoped(phase,
                      pltpu.VMEM((_KI, _D), jnp.float32),
                      pltpu.VMEM((_KI, _D), jnp.float32))


def _nlp_body(nlp_val, pos_nlp, rnlp, wnlp,
              onlp, ip, iw, ib, sem1, sem2):
    wid = lax.axis_index("s") * _NC + lax.axis_index("c")
    pltpu.sync_copy(wnlp, ib)

    for bb in range(_BPW):
        b = wid * _BPW + bb
        pltpu.sync_copy(rnlp.at[b].at[0], ip)
        for o in _chunk_offsets(_KN):
            sl = pl.ds(o, 16)
            iw[sl] = ib[sl] + b

        def phase(val, pos):
            c1 = pltpu.async_copy(nlp_val.at[b].at[ip], val, sem1)
            c2 = pltpu.async_copy(pos_nlp.at[ip], pos, sem2)
            c1.wait()
            c2.wait()
            _add_rows(_KN, val, pos)
            pltpu.async_copy(val, onlp.at[iw], sem1).wait()

        pl.run_scoped(phase,
                      pltpu.VMEM((_KN, _D), jnp.float32),
                      pltpu.VMEM((_KN, _D), jnp.float32))


@functools.cache
def _make_img_call():
  return pl.kernel(
    _img_body,
    out_type=jax.ShapeDtypeStruct((_B * _NR, _D), jnp.float32),
    mesh=plsc.VectorSubcoreMesh(core_axis_name="c", subcore_axis_name="s",
                                num_cores=_NC, num_subcores=_NS),
    compiler_params=pltpu.CompilerParams(use_tc_tiling_on_sc=True),
    scratch_types=[
        pltpu.VMEM((_KI,), jnp.int32),
        pltpu.VMEM((_KI,), jnp.int32),
        pltpu.VMEM((_KI,), jnp.int32),
        pltpu.VMEM((_KI,), jnp.int32),
        pltpu.SemaphoreType.DMA,
        pltpu.SemaphoreType.DMA,
    ],
  )


@functools.cache
def _make_nlp_call():
  return pl.kernel(
    _nlp_body,
    out_type=jax.ShapeDtypeStruct((_B * _NRN, _D), jnp.float32),
    mesh=plsc.VectorSubcoreMesh(core_axis_name="c", subcore_axis_name="s",
                                num_cores=_NC, num_subcores=_NS),
    compiler_params=pltpu.CompilerParams(use_tc_tiling_on_sc=True),
    scratch_types=[
        pltpu.VMEM((_KN,), jnp.int32),
        pltpu.VMEM((_KN,), jnp.int32),
        pltpu.VMEM((_KN,), jnp.int32),
        pltpu.SemaphoreType.DMA,
        pltpu.SemaphoreType.DMA,
    ],
  )


@jax.jit
def kernel(img_val, nlp_val, img_noise, nlp_remain_idx, nlp_masked_idx,
           nlp_revert_idx, pos_enc_2d):
    pe = jnp.asarray(_PE_NLP)
    wimg = jnp.asarray(_WIMG)
    wnlp = jnp.asarray(_WNLP)
    rnlp_pad = jnp.concatenate(
        [nlp_remain_idx, nlp_remain_idx[:, :_PAD_N]], axis=1)[:, None, :]
    onlp = _make_nlp_call()(nlp_val, pe, rnlp_pad, wnlp)

    rev, shuf, rimg_pad = _rank_call(
        img_noise[:, None, :], img_noise[:, :, None])
    rev = rev.reshape(_B, _N)
    shuf = shuf.reshape(_B, _N)
    masked_idx = shuf[:, _NR:]

    img_flat = img_val.transpose(1, 0, 2).reshape(_N * _B, _D)
    oimg = _make_img_call()(img_flat, pos_enc_2d, rimg_pad, wimg)
    img_remain = oimg.reshape(_NR, _B, _D).transpose(1, 0, 2)
    nlp_remain = onlp.reshape(_NRN, _B, _D).transpose(1, 0, 2)
    ones_r = jnp.ones((_B, _NR), jnp.float32)
    ones_m = jnp.ones((_B, _N - _NR), jnp.float32)
    ones_rev = jnp.ones((_B, _N), jnp.float32)
    return (img_remain, masked_idx, rev, ones_r, ones_m, ones_rev,
            nlp_remain, nlp_masked_idx, nlp_revert_idx)


# speedup vs baseline: 1.0418x; 1.0418x over previous
import functools

import numpy as np
import jax
import jax.numpy as jnp
from jax import lax
from jax.experimental import pallas as pl
from jax.experimental.pallas import tpu as pltpu
from jax.experimental.pallas import tpu_sc as plsc

_B = 64
_N = 196
_NR = 49
_LN = 512
_NRN = 76
_D = 768

_NC = 2
_NS = 16
_NW = _NC * _NS
_BPW = _B // _NW
_CH = _D // 16
_KI = 56
_KN = 80
_PAD_I = _KI - _NR
_PAD_N = _KN - _NRN


def _sinusoidal_pe(d_model, max_len):
    position = np.arange(max_len, dtype=np.float32)[:, None]
    div_term = np.exp(
        np.arange(0, d_model, 2, dtype=np.float32) * (-np.log(10000.0) / d_model))
    pe = np.zeros((max_len, d_model), dtype=np.float32)
    pe[:, 0::2] = np.sin(position * div_term)
    pe[:, 1::2] = np.cos(position * div_term)
    return pe


_PE_NLP = _sinusoidal_pe(_D, _LN)
_WIMG = (_B * np.concatenate([np.arange(_NR), np.arange(_KI - _NR)])
         ).astype(np.int32)
_WNLP = (_B * np.concatenate([np.arange(_NRN), np.arange(_KN - _NRN)])
         ).astype(np.int32)
_BPB = 8


def _rank_body(noise_ref, noiset_ref,
               rev_ref, shuf_ref, fimg_ref):
    g = pl.program_id(0)
    ii = lax.broadcasted_iota(jnp.int32, (_N, _N), 0)
    jj = lax.broadcasted_iota(jnp.int32, (_N, _N), 1)
    for i in range(_BPB):
        b = g * _BPB + i
        nrow = noise_ref[i]
        ncol = noiset_ref[i]
        m = ((nrow < ncol) | ((nrow == ncol) & (jj < ii))).astype(jnp.int32)
        rank_col = jnp.sum(m, axis=1, keepdims=True)
        rank_row = (_N - 1) - jnp.sum(m, axis=0, keepdims=True)
        rev_ref[i] = rank_row
        oneh = (rank_col == jj).astype(jnp.int32)
        shuf = jnp.sum(oneh * ii, axis=0, keepdims=True)
        shuf_ref[i] = shuf
        spad = jnp.concatenate([shuf[:, :_NR], shuf[:, :_PAD_I]], axis=1)
        fimg_ref[i] = spad * _B + b


_rank_call = pl.pallas_call(
    _rank_body,
    grid=(_B // _BPB,),
    in_specs=[
        pl.BlockSpec((_BPB, 1, _N), lambda b: (b, 0, 0)),
        pl.BlockSpec((_BPB, _N, 1), lambda b: (b, 0, 0)),
    ],
    out_specs=[
        pl.BlockSpec((_BPB, 1, _N), lambda b: (b, 0, 0)),
        pl.BlockSpec((_BPB, 1, _N), lambda b: (b, 0, 0)),
        pl.BlockSpec((_BPB, 1, _KI), lambda b: (b, 0, 0)),
    ],
    out_shape=[
        jax.ShapeDtypeStruct((_B, 1, _N), jnp.int32),
        jax.ShapeDtypeStruct((_B, 1, _N), jnp.int32),
        jax.ShapeDtypeStruct((_B, 1, _KI), jnp.int32),
    ],
)


def _chunk_offsets(n):
    offs = list(range(0, n - 15, 16))
    if offs[-1] + 16 < n:
        offs.append(n - 16)
    return offs


def _add_rows(k, val, pos):
    @pl.loop(0, k)
    def _addrow(r):
        for c in range(_CH):
            sl = pl.ds(c * 16, 16)
            plsc.addupdate(val.at[r, sl], pos[r, sl])


def _gather_body(img_flat, pos_img, nlp_val, pos_nlp, rimg, rnlp, wimg, wnlp,
                 oimg, onlp, ip_i, ipr_i, ip_n, iw_i, iw_n, ib_i, ib_n,
                 sem1, sem2):
    wid = lax.axis_index("s") * _NC + lax.axis_index("c")
    pltpu.sync_copy(wimg, ib_i)
    pltpu.sync_copy(wnlp, ib_n)

    def img_batch(b):
        for o in _chunk_offsets(_KI):
            sl = pl.ds(o, 16)
            iw_i[sl] = ib_i[sl] + b
            ipr_i[sl] = lax.shift_right_logical(ip_i[sl], 6)

        def phase(val, pos):
            c1 = pltpu.async_copy(img_flat.at[ip_i], val, sem1)
            c2 = pltpu.async_copy(pos_img.at[ipr_i], pos, sem2)
            c1.wait()
            c2.wait()
            _add_rows(_KI, val, pos)
            pltpu.async_copy(val, oimg.at[iw_i], sem1).wait()

        pl.run_scoped(phase,
                      pltpu.VMEM((_KI, _D), jnp.float32),
                      pltpu.VMEM((_KI, _D), jnp.float32))

    def nlp_batch(b):
        for o in _chunk_offsets(_KN):
            sl = pl.ds(o, 16)
            iw_n[sl] = ib_n[sl] + b

        def phase(val, pos):
            c1 = pltpu.async_copy(nlp_val.at[b].at[ip_n], val, sem1)
            c2 = pltpu.async_copy(pos_nlp.at[ip_n], pos, sem2)
            c1.wait()
            c2.wait()
            _add_rows(_KN, val, pos)
            pltpu.async_copy(val, onlp.at[iw_n], sem1).wait()

        pl.run_scoped(phase,
                      pltpu.VMEM((_KN, _D), jnp.float32),
                      pltpu.VMEM((_KN, _D), jnp.float32))

    for bb in range(_BPW):
        b = wid * _BPW + bb
        pltpu.sync_copy(rimg.at[b].at[0], ip_i)
        pltpu.sync_copy(rnlp.at[b].at[0], ip_n)
        img_batch(b)
        nlp_batch(b)


@functools.cache
def _make_gather_call():
  return pl.kernel(
    _gather_body,
    out_type=(
        jax.ShapeDtypeStruct((_B * _NR, _D), jnp.float32),
        jax.ShapeDtypeStruct((_B * _NRN, _D), jnp.float32),
    ),
    mesh=plsc.VectorSubcoreMesh(core_axis_name="c", subcore_axis_name="s",
                                num_cores=_NC, num_subcores=_NS),
    compiler_params=pltpu.CompilerParams(use_tc_tiling_on_sc=True),
    scratch_types=[
        pltpu.VMEM((_KI,), jnp.int32),
        pltpu.VMEM((_KI,), jnp.int32),
        pltpu.VMEM((_KN,), jnp.int32),
        pltpu.VMEM((_KI,), jnp.int32),
        pltpu.VMEM((_KN,), jnp.int32),
        pltpu.VMEM((_KI,), jnp.int32),
        pltpu.VMEM((_KN,), jnp.int32),
        pltpu.SemaphoreType.DMA,
        pltpu.SemaphoreType.DMA,
    ],
  )


@jax.jit
def kernel(img_val, nlp_val, img_noise, nlp_remain_idx, nlp_masked_idx,
           nlp_revert_idx, pos_enc_2d):
    pe = jnp.asarray(_PE_NLP)
    wimg = jnp.asarray(_WIMG)
    wnlp = jnp.asarray(_WNLP)
    rnlp_pad = jnp.concatenate(
        [nlp_remain_idx, nlp_remain_idx[:, :_PAD_N]], axis=1)[:, None, :]

    rev, shuf, rimg_pad = _rank_call(
        img_noise[:, None, :], img_noise[:, :, None])
    rev = rev.reshape(_B, _N)
    shuf = shuf.reshape(_B, _N)
    masked_idx = shuf[:, _NR:]

    img_flat = img_val.transpose(1, 0, 2).reshape(_N * _B, _D)
    oimg, onlp = _make_gather_call()(
        img_flat, pos_enc_2d, nlp_val, pe, rimg_pad, rnlp_pad, wimg, wnlp)
    img_remain = oimg.reshape(_NR, _B, _D).transpose(1, 0, 2)
    nlp_remain = onlp.reshape(_NRN, _B, _D).transpose(1, 0, 2)
    ones_r = jnp.ones((_B, _NR), jnp.float32)
    ones_m = jnp.ones((_B, _N - _NR), jnp.float32)
    ones_rev = jnp.ones((_B, _N), jnp.float32)
    return (img_remain, masked_idx, rev, ones_r, ones_m, ones_rev,
            nlp_remain, nlp_masked_idx, nlp_revert_idx)


# speedup vs baseline: 1.0642x; 1.0215x over previous
import functools

import numpy as np
import jax
import jax.numpy as jnp
from jax import lax
from jax.experimental import pallas as pl
from jax.experimental.pallas import tpu as pltpu
from jax.experimental.pallas import tpu_sc as plsc

_B = 64
_N = 196
_NR = 49
_LN = 512
_NRN = 76
_D = 768

_NC = 2
_NS = 16
_NW = _NC * _NS
_BPW = _B // _NW
_CH = _D // 16
_KI = 56
_KN = 80
_PAD_I = _KI - _NR
_PAD_N = _KN - _NRN


def _sinusoidal_pe(d_model, max_len):
    position = np.arange(max_len, dtype=np.float32)[:, None]
    div_term = np.exp(
        np.arange(0, d_model, 2, dtype=np.float32) * (-np.log(10000.0) / d_model))
    pe = np.zeros((max_len, d_model), dtype=np.float32)
    pe[:, 0::2] = np.sin(position * div_term)
    pe[:, 1::2] = np.cos(position * div_term)
    return pe


_PE_NLP = _sinusoidal_pe(_D, _LN)
_WIMG = (_B * np.concatenate([np.arange(_NR), np.arange(_KI - _NR)])
         ).astype(np.int32)
_WNLP = (_B * np.concatenate([np.arange(_NRN), np.arange(_KN - _NRN)])
         ).astype(np.int32)
_BPB = 8


def _rank_body(noise_ref, noiset_ref, nlpidx_ref,
               rev_ref, shuf_ref, fimg_ref, fnlp_ref):
    g = pl.program_id(0)
    ii = lax.broadcasted_iota(jnp.int32, (_N, _N), 0)
    jj = lax.broadcasted_iota(jnp.int32, (_N, _N), 1)
    for i in range(_BPB):
        b = g * _BPB + i
        nrow = noise_ref[i]
        ncol = noiset_ref[i]
        m = ((nrow < ncol) | ((nrow == ncol) & (jj < ii))).astype(jnp.int32)
        rank_col = jnp.sum(m, axis=1, keepdims=True)
        rank_row = (_N - 1) - jnp.sum(m, axis=0, keepdims=True)
        rev_ref[i] = rank_row
        oneh = (rank_col == jj).astype(jnp.int32)
        shuf = jnp.sum(oneh * ii, axis=0, keepdims=True)
        shuf_ref[i] = shuf
        spad = jnp.concatenate([shuf[:, :_NR], shuf[:, :_PAD_I]], axis=1)
        fimg_ref[i] = spad * _B + b
        nlpidx = nlpidx_ref[i]
        fnlp_ref[i] = jnp.concatenate([nlpidx, nlpidx[:, :_PAD_N]], axis=1)


_rank_call = pl.pallas_call(
    _rank_body,
    grid=(_B // _BPB,),
    in_specs=[
        pl.BlockSpec((_BPB, 1, _N), lambda b: (b, 0, 0)),
        pl.BlockSpec((_BPB, _N, 1), lambda b: (b, 0, 0)),
        pl.BlockSpec((_BPB, 1, _NRN), lambda b: (b, 0, 0)),
    ],
    out_specs=[
        pl.BlockSpec((_BPB, 1, _N), lambda b: (b, 0, 0)),
        pl.BlockSpec((_BPB, 1, _N), lambda b: (b, 0, 0)),
        pl.BlockSpec((_BPB, 1, _KI), lambda b: (b, 0, 0)),
        pl.BlockSpec((_BPB, 1, _KN), lambda b: (b, 0, 0)),
    ],
    out_shape=[
        jax.ShapeDtypeStruct((_B, 1, _N), jnp.int32),
        jax.ShapeDtypeStruct((_B, 1, _N), jnp.int32),
        jax.ShapeDtypeStruct((_B, 1, _KI), jnp.int32),
        jax.ShapeDtypeStruct((_B, 1, _KN), jnp.int32),
    ],
)


def _chunk_offsets(n):
    offs = list(range(0, n - 15, 16))
    if offs[-1] + 16 < n:
        offs.append(n - 16)
    return offs


def _add_rows(k, val, pos):
    @pl.loop(0, k)
    def _addrow(r):
        for c in range(_CH):
            sl = pl.ds(c * 16, 16)
            plsc.addupdate(val.at[r, sl], pos[r, sl])


def _gather_body(img_flat, pos_img, nlp_val, pos_nlp, rimg, rnlp, wimg, wnlp,
                 oimg, onlp, ip_i, ipr_i, ip_n, iw_i, iw_n, ib_i, ib_n,
                 sem1, sem2):
    wid = lax.axis_index("s") * _NC + lax.axis_index("c")
    pltpu.sync_copy(wimg, ib_i)
    pltpu.sync_copy(wnlp, ib_n)

    def img_batch(b):
        for o in _chunk_offsets(_KI):
            sl = pl.ds(o, 16)
            iw_i[sl] = ib_i[sl] + b
            ipr_i[sl] = lax.shift_right_logical(ip_i[sl], 6)

        def phase(val, pos):
            c1 = pltpu.async_copy(img_flat.at[ip_i], val, sem1)
            c2 = pltpu.async_copy(pos_img.at[ipr_i], pos, sem2)
            c1.wait()
            c2.wait()
            _add_rows(_KI, val, pos)
            pltpu.async_copy(val, oimg.at[iw_i], sem1).wait()

        pl.run_scoped(phase,
                      pltpu.VMEM((_KI, _D), jnp.float32),
                      pltpu.VMEM((_KI, _D), jnp.float32))

    def nlp_batch(b):
        for o in _chunk_offsets(_KN):
            sl = pl.ds(o, 16)
            iw_n[sl] = ib_n[sl] + b

        def phase(val, pos):
            c1 = pltpu.async_copy(nlp_val.at[b].at[ip_n], val, sem1)
            c2 = pltpu.async_copy(pos_nlp.at[ip_n], pos, sem2)
            c1.wait()
            c2.wait()
            _add_rows(_KN, val, pos)
            pltpu.async_copy(val, onlp.at[iw_n], sem1).wait()

        pl.run_scoped(phase,
                      pltpu.VMEM((_KN, _D), jnp.float32),
                      pltpu.VMEM((_KN, _D), jnp.float32))

    for bb in range(_BPW):
        b = wid * _BPW + bb
        pltpu.sync_copy(rimg.at[b].at[0], ip_i)
        pltpu.sync_copy(rnlp.at[b].at[0], ip_n)
        img_batch(b)
        nlp_batch(b)


@functools.cache
def _make_gather_call():
  return pl.kernel(
    _gather_body,
    out_type=(
        jax.ShapeDtypeStruct((_B * _NR, _D), jnp.float32),
        jax.ShapeDtypeStruct((_B * _NRN, _D), jnp.float32),
    ),
    mesh=plsc.VectorSubcoreMesh(core_axis_name="c", subcore_axis_name="s",
                                num_cores=_NC, num_subcores=_NS),
    compiler_params=pltpu.CompilerParams(use_tc_tiling_on_sc=True),
    scratch_types=[
        pltpu.VMEM((_KI,), jnp.int32),
        pltpu.VMEM((_KI,), jnp.int32),
        pltpu.VMEM((_KN,), jnp.int32),
        pltpu.VMEM((_KI,), jnp.int32),
        pltpu.VMEM((_KN,), jnp.int32),
        pltpu.VMEM((_KI,), jnp.int32),
        pltpu.VMEM((_KN,), jnp.int32),
        pltpu.SemaphoreType.DMA,
        pltpu.SemaphoreType.DMA,
    ],
  )


@jax.jit
def kernel(img_val, nlp_val, img_noise, nlp_remain_idx, nlp_masked_idx,
           nlp_revert_idx, pos_enc_2d):
    pe = jnp.asarray(_PE_NLP)
    wimg = jnp.asarray(_WIMG)
    wnlp = jnp.asarray(_WNLP)
    rev, shuf, rimg_pad, rnlp_pad = _rank_call(
        img_noise[:, None, :], img_noise[:, :, None],
        nlp_remain_idx[:, None, :])
    rev = rev.reshape(_B, _N)
    shuf = shuf.reshape(_B, _N)
    masked_idx = shuf[:, _NR:]

    img_flat = img_val.transpose(1, 0, 2).reshape(_N * _B, _D)
    oimg, onlp = _make_gather_call()(
        img_flat, pos_enc_2d, nlp_val, pe, rimg_pad, rnlp_pad, wimg, wnlp)
    img_remain = oimg.reshape(_NR, _B, _D).transpose(1, 0, 2)
    nlp_remain = onlp.reshape(_NRN, _B, _D).transpose(1, 0, 2)
    ones_r = jnp.ones((_B, _NR), jnp.float32)
    ones_m = jnp.ones((_B, _N - _NR), jnp.float32)
    ones_rev = jnp.ones((_B, _N), jnp.float32)
    return (img_remain, masked_idx, rev, ones_r, ones_m, ones_rev,
            nlp_remain, nlp_masked_idx, nlp_revert_idx)


# speedup vs baseline: 1.0733x; 1.0086x over previous
import functools

import numpy as np
import jax
import jax.numpy as jnp
from jax import lax
from jax.experimental import pallas as pl
from jax.experimental.pallas import tpu as pltpu
from jax.experimental.pallas import tpu_sc as plsc

_B = 64
_N = 196
_NR = 49
_LN = 512
_NRN = 76
_D = 768

_NC = 2
_NS = 16
_NW = _NC * _NS
_BPW = _B // _NW
_CH = _D // 16
_KI = 56
_KN = 80
_PAD_I = _KI - _NR
_PAD_N = _KN - _NRN


def _sinusoidal_pe(d_model, max_len):
    position = np.arange(max_len, dtype=np.float32)[:, None]
    div_term = np.exp(
        np.arange(0, d_model, 2, dtype=np.float32) * (-np.log(10000.0) / d_model))
    pe = np.zeros((max_len, d_model), dtype=np.float32)
    pe[:, 0::2] = np.sin(position * div_term)
    pe[:, 1::2] = np.cos(position * div_term)
    return pe


_PE_NLP = _sinusoidal_pe(_D, _LN)
_WIMG = (_B * np.concatenate([np.arange(_NR), np.arange(_KI - _NR)])
         ).astype(np.int32)
_WNLP = (_B * np.concatenate([np.arange(_NRN), np.arange(_KN - _NRN)])
         ).astype(np.int32)
_BPB = 8


def _rank_body(noise_ref, noiset_ref, nlpidx_ref,
               rev_ref, shuf_ref, fimg_ref, fnlp_ref):
    g = pl.program_id(0)
    ii = lax.broadcasted_iota(jnp.int32, (_N, _N), 0)
    jj = lax.broadcasted_iota(jnp.int32, (_N, _N), 1)
    for i in range(_BPB):
        b = g * _BPB + i
        nrow = noise_ref[i]
        ncol = noiset_ref[i]
        m = ((nrow < ncol) | ((nrow == ncol) & (jj < ii))).astype(jnp.int32)
        rank_col = jnp.sum(m, axis=1, keepdims=True)
        rank_row = (_N - 1) - jnp.sum(m, axis=0, keepdims=True)
        rev_ref[i] = rank_row
        oneh = (rank_col == jj).astype(jnp.int32)
        shuf = jnp.sum(oneh * ii, axis=0, keepdims=True)
        shuf_ref[i] = shuf
        spad = jnp.concatenate([shuf[:, :_NR], shuf[:, :_PAD_I]], axis=1)
        fimg_ref[i] = spad * _B + b
        nlpidx = nlpidx_ref[i]
        fnlp_ref[i] = jnp.concatenate([nlpidx, nlpidx[:, :_PAD_N]], axis=1)


_rank_call = pl.pallas_call(
    _rank_body,
    grid=(_B // _BPB,),
    in_specs=[
        pl.BlockSpec((_BPB, 1, _N), lambda b: (b, 0, 0)),
        pl.BlockSpec((_BPB, _N, 1), lambda b: (b, 0, 0)),
        pl.BlockSpec((_BPB, 1, _NRN), lambda b: (b, 0, 0)),
    ],
    out_specs=[
        pl.BlockSpec((_BPB, 1, _N), lambda b: (b, 0, 0)),
        pl.BlockSpec((_BPB, 1, _N), lambda b: (b, 0, 0)),
        pl.BlockSpec((_BPB, 1, _KI), lambda b: (b, 0, 0)),
        pl.BlockSpec((_BPB, 1, _KN), lambda b: (b, 0, 0)),
    ],
    out_shape=[
        jax.ShapeDtypeStruct((_B, 1, _N), jnp.int32),
        jax.ShapeDtypeStruct((_B, 1, _N), jnp.int32),
        jax.ShapeDtypeStruct((_B, 1, _KI), jnp.int32),
        jax.ShapeDtypeStruct((_B, 1, _KN), jnp.int32),
    ],
)


def _chunk_offsets(n):
    offs = list(range(0, n - 15, 16))
    if offs[-1] + 16 < n:
        offs.append(n - 16)
    return offs


def _add_rows(k, val, pos, r0=0):
    @pl.loop(0, k)
    def _addrow(r):
        for c in range(_CH):
            sl = pl.ds(c * 16, 16)
            plsc.addupdate(val.at[r0 + r, sl], pos[r, sl])


def _two_phase(mk_vsrc, ptab, pidx, out, iw, h1, h2, sem1, sem2, val,
               pos_a, pos_b):
    c1a = pltpu.async_copy(mk_vsrc(0, h1), val.at[pl.ds(0, h1)], sem1)
    c2a = pltpu.async_copy(ptab.at[pidx.at[pl.ds(0, h1)]], pos_a, sem2)
    c1b = pltpu.async_copy(mk_vsrc(h1, h2), val.at[pl.ds(h1, h2)], sem1)
    c2b = pltpu.async_copy(ptab.at[pidx.at[pl.ds(h1, h2)]], pos_b, sem2)
    c1a.wait()
    c2a.wait()
    _add_rows(h1, val, pos_a)
    c1b.wait()
    c2b.wait()
    _add_rows(h2, val, pos_b, r0=h1)
    pltpu.async_copy(val, out.at[iw], sem1).wait()


def _gather_body(img_flat, pos_img, nlp_val, pos_nlp, rimg, rnlp, wimg, wnlp,
                 oimg, onlp, ip_i, ipr_i, ip_n, iw_i, iw_n, ib_i, ib_n,
                 sem1, sem2):
    wid = lax.axis_index("s") * _NC + lax.axis_index("c")
    pltpu.sync_copy(wimg, ib_i)
    pltpu.sync_copy(wnlp, ib_n)

    def img_batch(b):
        for o in _chunk_offsets(_KI):
            sl = pl.ds(o, 16)
            iw_i[sl] = ib_i[sl] + b
            ipr_i[sl] = lax.shift_right_logical(ip_i[sl], 6)

        h1, h2 = 32, _KI - 32

        def mk_vsrc(lo, n):
            return img_flat.at[ip_i.at[pl.ds(lo, n)]]

        def phase(val, pos_a, pos_b):
            _two_phase(mk_vsrc, pos_img, ipr_i, oimg, iw_i, h1, h2,
                       sem1, sem2, val, pos_a, pos_b)

        pl.run_scoped(phase,
                      pltpu.VMEM((_KI, _D), jnp.float32),
                      pltpu.VMEM((h1, _D), jnp.float32),
                      pltpu.VMEM((h2, _D), jnp.float32))

    def nlp_batch(b):
        for o in _chunk_offsets(_KN):
            sl = pl.ds(o, 16)
            iw_n[sl] = ib_n[sl] + b

        h1, h2 = 40, _KN - 40

        def mk_vsrc(lo, n):
            return nlp_val.at[b].at[ip_n.at[pl.ds(lo, n)]]

        def phase(val, pos_a, pos_b):
            _two_phase(mk_vsrc, pos_nlp, ip_n, onlp, iw_n, h1, h2,
                       sem1, sem2, val, pos_a, pos_b)

        pl.run_scoped(phase,
                      pltpu.VMEM((_KN, _D), jnp.float32),
                      pltpu.VMEM((h1, _D), jnp.float32),
                      pltpu.VMEM((h2, _D), jnp.float32))

    for bb in range(_BPW):
        b = wid * _BPW + bb
        pltpu.sync_copy(rimg.at[b].at[0], ip_i)
        pltpu.sync_copy(rnlp.at[b].at[0], ip_n)
        img_batch(b)
        nlp_batch(b)


@functools.cache
def _make_gather_call():
  return pl.kernel(
    _gather_body,
    out_type=(
        jax.ShapeDtypeStruct((_B * _NR, _D), jnp.float32),
        jax.ShapeDtypeStruct((_B * _NRN, _D), jnp.float32),
    ),
    mesh=plsc.VectorSubcoreMesh(core_axis_name="c", subcore_axis_name="s",
                                num_cores=_NC, num_subcores=_NS),
    compiler_params=pltpu.CompilerParams(use_tc_tiling_on_sc=True),
    scratch_types=[
        pltpu.VMEM((_KI,), jnp.int32),
        pltpu.VMEM((_KI,), jnp.int32),
        pltpu.VMEM((_KN,), jnp.int32),
        pltpu.VMEM((_KI,), jnp.int32),
        pltpu.VMEM((_KN,), jnp.int32),
        pltpu.VMEM((_KI,), jnp.int32),
        pltpu.VMEM((_KN,), jnp.int32),
        pltpu.SemaphoreType.DMA,
        pltpu.SemaphoreType.DMA,
    ],
  )


@jax.jit
def kernel(img_val, nlp_val, img_noise, nlp_remain_idx, nlp_masked_idx,
           nlp_revert_idx, pos_enc_2d):
    pe = jnp.asarray(_PE_NLP)
    wimg = jnp.asarray(_WIMG)
    wnlp = jnp.asarray(_WNLP)
    rev, shuf, rimg_pad, rnlp_pad = _rank_call(
        img_noise[:, None, :], img_noise[:, :, None],
        nlp_remain_idx[:, None, :])
    rev = rev.reshape(_B, _N)
    shuf = shuf.reshape(_B, _N)
    masked_idx = shuf[:, _NR:]

    img_flat = img_val.transpose(1, 0, 2).reshape(_N * _B, _D)
    oimg, onlp = _make_gather_call()(
        img_flat, pos_enc_2d, nlp_val, pe, rimg_pad, rnlp_pad, wimg, wnlp)
    img_remain = oimg.reshape(_NR, _B, _D).transpose(1, 0, 2)
    nlp_remain = onlp.reshape(_NRN, _B, _D).transpose(1, 0, 2)
    ones_r = jnp.ones((_B, _NR), jnp.float32)
    ones_m = jnp.ones((_B, _N - _NR), jnp.float32)
    ones_rev = jnp.ones((_B, _N), jnp.float32)
    return (img_remain, masked_idx, rev, ones_r, ones_m, ones_rev,
            nlp_remain, nlp_masked_idx, nlp_revert_idx)


# speedup vs baseline: 1.0863x; 1.0121x over previous
import functools

import numpy as np
import jax
import jax.numpy as jnp
from jax import lax
from jax.experimental import pallas as pl
from jax.experimental.pallas import tpu as pltpu
from jax.experimental.pallas import tpu_sc as plsc

_B = 64
_N = 196
_NR = 49
_LN = 512
_NRN = 76
_D = 768

_NC = 2
_NS = 16
_NW = _NC * _NS
_BPW = _B // _NW
_CH = _D // 16
_KI = 56
_KN = 80
_PAD_I = _KI - _NR
_PAD_N = _KN - _NRN


def _sinusoidal_pe(d_model, max_len):
    position = np.arange(max_len, dtype=np.float32)[:, None]
    div_term = np.exp(
        np.arange(0, d_model, 2, dtype=np.float32) * (-np.log(10000.0) / d_model))
    pe = np.zeros((max_len, d_model), dtype=np.float32)
    pe[:, 0::2] = np.sin(position * div_term)
    pe[:, 1::2] = np.cos(position * div_term)
    return pe


_PE_NLP = _sinusoidal_pe(_D, _LN)
_WIMG = (_B * np.concatenate([np.arange(_NR), np.arange(_KI - _NR)])
         ).astype(np.int32)
_WNLP = (_B * np.concatenate([np.arange(_NRN), np.arange(_KN - _NRN)])
         ).astype(np.int32)
_BPB = 8


def _rank_body(noise_ref, nlpidx_ref,
               rev_ref, shuf_ref, fimg_ref, fnlp_ref):
    g = pl.program_id(0)
    ii = lax.broadcasted_iota(jnp.int32, (_N, _N), 0)
    jj = lax.broadcasted_iota(jnp.int32, (_N, _N), 1)
    for i in range(_BPB):
        b = g * _BPB + i
        nrow = noise_ref[i]
        ncol = jnp.transpose(nrow, (1, 0))
        m = ((nrow < ncol) | ((nrow == ncol) & (jj < ii))).astype(jnp.int32)
        rank_col = jnp.sum(m, axis=1, keepdims=True)
        rank_row = (_N - 1) - jnp.sum(m, axis=0, keepdims=True)
        rev_ref[i] = rank_row
        oneh = (rank_col == jj).astype(jnp.int32)
        shuf = jnp.sum(oneh * ii, axis=0, keepdims=True)
        shuf_ref[i] = shuf
        spad = jnp.concatenate([shuf[:, :_NR], shuf[:, :_PAD_I]], axis=1)
        fimg_ref[i] = spad * _B + b
        nlpidx = nlpidx_ref[i]
        fnlp_ref[i] = jnp.concatenate([nlpidx, nlpidx[:, :_PAD_N]], axis=1)


_rank_call = pl.pallas_call(
    _rank_body,
    grid=(_B // _BPB,),
    in_specs=[
        pl.BlockSpec((_BPB, 1, _N), lambda b: (b, 0, 0)),
        pl.BlockSpec((_BPB, 1, _NRN), lambda b: (b, 0, 0)),
    ],
    out_specs=[
        pl.BlockSpec((_BPB, 1, _N), lambda b: (b, 0, 0)),
        pl.BlockSpec((_BPB, 1, _N), lambda b: (b, 0, 0)),
        pl.BlockSpec((_BPB, 1, _KI), lambda b: (b, 0, 0)),
        pl.BlockSpec((_BPB, 1, _KN), lambda b: (b, 0, 0)),
    ],
    out_shape=[
        jax.ShapeDtypeStruct((_B, 1, _N), jnp.int32),
        jax.ShapeDtypeStruct((_B, 1, _N), jnp.int32),
        jax.ShapeDtypeStruct((_B, 1, _KI), jnp.int32),
        jax.ShapeDtypeStruct((_B, 1, _KN), jnp.int32),
    ],
)


def _chunk_offsets(n):
    offs = list(range(0, n - 15, 16))
    if offs[-1] + 16 < n:
        offs.append(n - 16)
    return offs


def _add_rows(k, val, pos, r0=0):
    @pl.loop(0, k)
    def _addrow(r):
        for c in range(_CH):
            sl = pl.ds(c * 16, 16)
            plsc.addupdate(val.at[r0 + r, sl], pos[r, sl])


def _two_phase(mk_vsrc, ptab, pidx, out, iw, h1, h2, sem1, sem2, val,
               pos_a, pos_b):
    c1a = pltpu.async_copy(mk_vsrc(0, h1), val.at[pl.ds(0, h1)], sem1)
    c2a = pltpu.async_copy(ptab.at[pidx.at[pl.ds(0, h1)]], pos_a, sem2)
    c1b = pltpu.async_copy(mk_vsrc(h1, h2), val.at[pl.ds(h1, h2)], sem1)
    c2b = pltpu.async_copy(ptab.at[pidx.at[pl.ds(h1, h2)]], pos_b, sem2)
    c1a.wait()
    c2a.wait()
    _add_rows(h1, val, pos_a)
    c1b.wait()
    c2b.wait()
    _add_rows(h2, val, pos_b, r0=h1)
    pltpu.async_copy(val, out.at[iw], sem1).wait()


def _gather_body(img_flat, pos_img, nlp_val, pos_nlp, rimg, rnlp, wimg, wnlp,
                 oimg, onlp, ip_i, ipr_i, ip_n, iw_i, iw_n, ib_i, ib_n,
                 sem1, sem2):
    wid = lax.axis_index("s") * _NC + lax.axis_index("c")
    pltpu.sync_copy(wimg, ib_i)
    pltpu.sync_copy(wnlp, ib_n)

    def img_batch(b):
        for o in _chunk_offsets(_KI):
            sl = pl.ds(o, 16)
            iw_i[sl] = ib_i[sl] + b
            ipr_i[sl] = lax.shift_right_logical(ip_i[sl], 6)

        h1, h2 = 32, _KI - 32

        def mk_vsrc(lo, n):
            return img_flat.at[ip_i.at[pl.ds(lo, n)]]

        def phase(val, pos_a, pos_b):
            _two_phase(mk_vsrc, pos_img, ipr_i, oimg, iw_i, h1, h2,
                       sem1, sem2, val, pos_a, pos_b)

        pl.run_scoped(phase,
                      pltpu.VMEM((_KI, _D), jnp.float32),
                      pltpu.VMEM((h1, _D), jnp.float32),
                      pltpu.VMEM((h2, _D), jnp.float32))

    def nlp_batch(b):
        for o in _chunk_offsets(_KN):
            sl = pl.ds(o, 16)
            iw_n[sl] = ib_n[sl] + b

        h1, h2 = 40, _KN - 40

        def mk_vsrc(lo, n):
            return nlp_val.at[b].at[ip_n.at[pl.ds(lo, n)]]

        def phase(val, pos_a, pos_b):
            _two_phase(mk_vsrc, pos_nlp, ip_n, onlp, iw_n, h1, h2,
                       sem1, sem2, val, pos_a, pos_b)

        pl.run_scoped(phase,
                      pltpu.VMEM((_KN, _D), jnp.float32),
                      pltpu.VMEM((h1, _D), jnp.float32),
                      pltpu.VMEM((h2, _D), jnp.float32))

    for bb in range(_BPW):
        b = wid * _BPW + bb
        pltpu.sync_copy(rimg.at[b].at[0], ip_i)
        pltpu.sync_copy(rnlp.at[b].at[0], ip_n)
        img_batch(b)
        nlp_batch(b)


@functools.cache
def _make_gather_call():
  return pl.kernel(
    _gather_body,
    out_type=(
        jax.ShapeDtypeStruct((_B * _NR, _D), jnp.float32),
        jax.ShapeDtypeStruct((_B * _NRN, _D), jnp.float32),
    ),
    mesh=plsc.VectorSubcoreMesh(core_axis_name="c", subcore_axis_name="s",
                                num_cores=_NC, num_subcores=_NS),
    compiler_params=pltpu.CompilerParams(use_tc_tiling_on_sc=True),
    scratch_types=[
        pltpu.VMEM((_KI,), jnp.int32),
        pltpu.VMEM((_KI,), jnp.int32),
        pltpu.VMEM((_KN,), jnp.int32),
        pltpu.VMEM((_KI,), jnp.int32),
        pltpu.VMEM((_KN,), jnp.int32),
        pltpu.VMEM((_KI,), jnp.int32),
        pltpu.VMEM((_KN,), jnp.int32),
        pltpu.SemaphoreType.DMA,
        pltpu.SemaphoreType.DMA,
    ],
  )


@jax.jit
def kernel(img_val, nlp_val, img_noise, nlp_remain_idx, nlp_masked_idx,
           nlp_revert_idx, pos_enc_2d):
    pe = jnp.asarray(_PE_NLP)
    wimg = jnp.asarray(_WIMG)
    wnlp = jnp.asarray(_WNLP)
    rev, shuf, rimg_pad, rnlp_pad = _rank_call(
        img_noise[:, None, :], nlp_remain_idx[:, None, :])
    rev = rev.reshape(_B, _N)
    shuf = shuf.reshape(_B, _N)
    masked_idx = shuf[:, _NR:]

    img_flat = img_val.transpose(1, 0, 2).reshape(_N * _B, _D)
    oimg, onlp = _make_gather_call()(
        img_flat, pos_enc_2d, nlp_val, pe, rimg_pad, rnlp_pad, wimg, wnlp)
    img_remain = oimg.reshape(_NR, _B, _D).transpose(1, 0, 2)
    nlp_remain = onlp.reshape(_NRN, _B, _D).transpose(1, 0, 2)
    ones_r = jnp.ones((_B, _NR), jnp.float32)
    ones_m = jnp.ones((_B, _N - _NR), jnp.float32)
    ones_rev = jnp.ones((_B, _N), jnp.float32)
    return (img_remain, masked_idx, rev, ones_r, ones_m, ones_rev,
            nlp_remain, nlp_masked_idx, nlp_revert_idx)
